# baseline (speedup 1.0000x reference)
import functools

import jax
import jax.numpy as jnp
from jax import lax
from jax.experimental import pallas as pl
from jax.experimental.pallas import tpu as pltpu
from jax.experimental.pallas import tpu_sc as plsc

B = 4096
D = 64
NC = 2
NS = 16
NW = NC * NS
B_PER_W = B // NW

_sc_mesh = plsc.VectorSubcoreMesh(core_axis_name="c", subcore_axis_name="s")


@functools.partial(
    pl.kernel,
    mesh=_sc_mesh,
    compiler_params=pltpu.CompilerParams(use_tc_tiling_on_sc=False),
    out_type=[
        jax.ShapeDtypeStruct((B, D), jnp.float32),
        jax.ShapeDtypeStruct((B, D), jnp.float32),
    ],
    scratch_types=[
        pltpu.VMEM((B_PER_W,), jnp.int32),
        pltpu.VMEM((B_PER_W, D), jnp.float32),
        pltpu.VMEM((B_PER_W,), jnp.int32),
        pltpu.VMEM((B_PER_W, D), jnp.float32),
        pltpu.SemaphoreType.DMA,
        pltpu.SemaphoreType.DMA,
    ],
)
def _sc_gather2(node_hbm, ctx_hbm, wtab_hbm, ctab_hbm, ew_hbm, ec_hbm,
                idx_w, rows_w, idx_c, rows_c, sem_w, sem_c):
    wid = lax.axis_index("s") * NC + lax.axis_index("c")
    base = wid * B_PER_W
    pltpu.sync_copy(node_hbm.at[pl.ds(base, B_PER_W)], idx_w)
    pltpu.sync_copy(ctx_hbm.at[pl.ds(base, B_PER_W)], idx_c)
    cp_w = pltpu.async_copy(wtab_hbm.at[idx_w], rows_w, sem_w)
    cp_c = pltpu.async_copy(ctab_hbm.at[idx_c], rows_c, sem_c)
    cp_w.wait()
    cp_c.wait()
    pltpu.sync_copy(rows_w, ew_hbm.at[pl.ds(base, B_PER_W)])
    pltpu.sync_copy(rows_c, ec_hbm.at[pl.ds(base, B_PER_W)])


CB = 512


def _score_body(ec_ref, ew_ref, o_ref):
    i = pl.program_id(0)
    s = lax.dot_general(
        ec_ref[...], ew_ref[...],
        dimension_numbers=(((1,), (1,)), ((), ())),
        preferred_element_type=jnp.float32,
    )
    val = jnp.sum(jnp.maximum(-s, 0.0) + jnp.log1p(jnp.exp(-jnp.abs(s))))

    @pl.when(i == 0)
    def _init():
        o_ref[0, 0] = 0.0

    o_ref[0, 0] += val


_score = pl.pallas_call(
    _score_body,
    grid=(B // CB,),
    in_specs=[
        pl.BlockSpec((CB, D), lambda i: (i, 0)),
        pl.BlockSpec((B, D), lambda i: (0, 0)),
    ],
    out_specs=pl.BlockSpec(
        (1, 1), lambda i: (0, 0), memory_space=pltpu.SMEM),
    out_shape=jax.ShapeDtypeStruct((1, 1), jnp.float32),
)


def kernel(node, context_positions, word_embedding, context_embedding):
    embed_word, embed_context = _sc_gather2(
        node, context_positions, word_embedding, context_embedding)
    obj = _score(embed_context, embed_word)
    return obj[0, 0]


# speedup vs baseline: 1.3180x; 1.3180x over previous
import functools

import jax
import jax.numpy as jnp
from jax import lax
from jax.experimental import pallas as pl
from jax.experimental.pallas import tpu as pltpu
from jax.experimental.pallas import tpu_sc as plsc

NUM_NODES = 1000000
B = 4096
D = 64
GR = 32
LB = 4096
PG = NUM_NODES // LB + 1
NG_PAD = PG * (LB // GR)
NC = 2
NS = 16
NW = NC * NS
B_PER_W = B // NW
CH = 32


def _pack_body(xt_ref, out_ref):
    xt = lax.transpose(xt_ref[...], (1, 0))
    y = xt.astype(jnp.bfloat16)
    wt = pltpu.bitcast(y, jnp.int32)
    wt3 = wt.reshape(LB // 4, 2, D)
    ev = wt3[:, 0, :]
    od = wt3[:, 1, :]
    out_ref[:, :, 0:64] = ev.reshape(LB // GR, 8, D)
    out_ref[:, :, 64:128] = od.reshape(LB // GR, 8, D)


_pack = pl.pallas_call(
    _pack_body,
    grid=(PG,),
    in_specs=[pl.BlockSpec((D, LB), lambda i: (0, i))],
    out_specs=pl.BlockSpec((LB // GR, 8, 128), lambda i: (i, 0, 0)),
    out_shape=jax.ShapeDtypeStruct((NG_PAD, 8, 128), jnp.int32),
)


_sc_mesh = plsc.VectorSubcoreMesh(core_axis_name="c", subcore_axis_name="s")


@functools.partial(
    pl.kernel,
    mesh=_sc_mesh,
    out_type=[
        jax.ShapeDtypeStruct((B, 8, 128), jnp.int32),
        jax.ShapeDtypeStruct((B, 8, 128), jnp.int32),
    ],
    scratch_types=[
        pltpu.VMEM((B_PER_W,), jnp.int32),
        pltpu.VMEM((B_PER_W,), jnp.int32),
        pltpu.VMEM((CH, 8, 128), jnp.int32),
        pltpu.VMEM((CH, 8, 128), jnp.int32),
        pltpu.SemaphoreType.DMA,
        pltpu.SemaphoreType.DMA,
    ],
)
def _sc_gather2(gw_hbm, gc_hbm, wtab_hbm, ctab_hbm, ew_hbm, ec_hbm,
                idx_w, idx_c, rows_w, rows_c, sem_w, sem_c):
    wid = lax.axis_index("s") * NC + lax.axis_index("c")
    base = wid * B_PER_W
    pltpu.sync_copy(gw_hbm.at[pl.ds(base, B_PER_W)], idx_w)
    pltpu.sync_copy(gc_hbm.at[pl.ds(base, B_PER_W)], idx_c)
    for h in range(B_PER_W // CH):
        cp_w = pltpu.async_copy(
            wtab_hbm.at[idx_w.at[pl.ds(h * CH, CH)]], rows_w, sem_w)
        cp_c = pltpu.async_copy(
            ctab_hbm.at[idx_c.at[pl.ds(h * CH, CH)]], rows_c, sem_c)
        cp_w.wait()
        cp_c.wait()
        pltpu.sync_copy(rows_w, ew_hbm.at[pl.ds(base + h * CH, CH)])
        pltpu.sync_copy(rows_c, ec_hbm.at[pl.ds(base + h * CH, CH)])


SB = 512


def _sel_body(ew32_ref, ec32_ref, selw_ref, selc_ref, ew_ref, ec_ref):
    def pick(x_ref, sel_ref):
        sel = sel_ref[...]
        acc = jnp.zeros((SB, D), jnp.float32)
        hi_mask = jnp.int32(-65536)
        for u in range(8):
            for hf in range(2):
                y = x_ref[:, u, 64 * hf:64 * hf + 64]
                f_lo = lax.bitcast_convert_type(
                    lax.shift_left(y, 16), jnp.float32)
                f_hi = lax.bitcast_convert_type(
                    lax.bitwise_and(y, hi_mask), jnp.float32)
                q = 4 * u + 2 * hf
                m_lo = (sel == q).astype(jnp.float32)
                m_hi = (sel == (q + 1)).astype(jnp.float32)
                acc += f_lo * m_lo + f_hi * m_hi
        return acc

    ew_ref[...] = pick(ew32_ref, selw_ref)
    ec_ref[...] = pick(ec32_ref, selc_ref)


_select = pl.pallas_call(
    _sel_body,
    grid=(B // SB,),
    in_specs=[
        pl.BlockSpec((SB, 8, 128), lambda i: (i, 0, 0)),
        pl.BlockSpec((SB, 8, 128), lambda i: (i, 0, 0)),
        pl.BlockSpec((SB, 1), lambda i: (i, 0)),
        pl.BlockSpec((SB, 1), lambda i: (i, 0)),
    ],
    out_specs=[
        pl.BlockSpec((SB, D), lambda i: (i, 0)),
        pl.BlockSpec((SB, D), lambda i: (i, 0)),
    ],
    out_shape=[
        jax.ShapeDtypeStruct((B, D), jnp.float32),
        jax.ShapeDtypeStruct((B, D), jnp.float32),
    ],
)


CB = 512


def _score_body(ec_ref, ew_ref, o_ref):
    i = pl.program_id(0)
    s = lax.dot_general(
        ec_ref[...], ew_ref[...],
        dimension_numbers=(((1,), (1,)), ((), ())),
        preferred_element_type=jnp.float32,
    )
    val = jnp.sum(jnp.maximum(-s, 0.0) + jnp.log1p(jnp.exp(-jnp.abs(s))))

    @pl.when(i == 0)
    def _init():
        o_ref[0, 0] = 0.0

    o_ref[0, 0] += val


_score = pl.pallas_call(
    _score_body,
    grid=(B // CB,),
    in_specs=[
        pl.BlockSpec((CB, D), lambda i: (i, 0)),
        pl.BlockSpec((B, D), lambda i: (0, 0)),
    ],
    out_specs=pl.BlockSpec(
        (1, 1), lambda i: (0, 0), memory_space=pltpu.SMEM),
    out_shape=jax.ShapeDtypeStruct((1, 1), jnp.float32),
)


def kernel(node, context_positions, word_embedding, context_embedding):
    wtab = _pack(word_embedding.T)
    ctab = _pack(context_embedding.T)
    g_w = node // GR
    g_c = context_positions // GR
    sel_w = (node % GR).reshape(B, 1)
    sel_c = (context_positions % GR).reshape(B, 1)
    ew32, ec32 = _sc_gather2(g_w, g_c, wtab, ctab)
    embed_word, embed_context = _select(ew32, ec32, sel_w, sel_c)
    obj = _score(embed_context, embed_word)
    return obj[0, 0]


# speedup vs baseline: 1.5201x; 1.1534x over previous
import functools

import jax
import jax.numpy as jnp
from jax import lax
from jax.experimental import pallas as pl
from jax.experimental.pallas import tpu as pltpu
from jax.experimental.pallas import tpu_sc as plsc

NUM_NODES = 1000000
B = 4096
D = 64
GR = 32
LB = 4096
PG = NUM_NODES // LB + 1
NG_PAD = PG * (LB // GR)
NC = 2
NS = 16
NW = NC * NS
B_PER_W = B // NW
CH = 32


def _pack_body(xt_ref, out_ref):
    yb = xt_ref[...].astype(jnp.bfloat16)
    y = lax.transpose(yb, (1, 0))
    wt = pltpu.bitcast(y, jnp.int32)
    wt3 = wt.reshape(LB // 4, 2, D)
    ev = wt3[:, 0, :]
    od = wt3[:, 1, :]
    out_ref[:, :, 0:64] = ev.reshape(LB // GR, 8, D)
    out_ref[:, :, 64:128] = od.reshape(LB // GR, 8, D)


_pack = pl.pallas_call(
    _pack_body,
    grid=(PG,),
    in_specs=[pl.BlockSpec((D, LB), lambda i: (0, i))],
    out_specs=pl.BlockSpec((LB // GR, 8, 128), lambda i: (i, 0, 0)),
    out_shape=jax.ShapeDtypeStruct((NG_PAD, 8, 128), jnp.int32),
)


_sc_mesh = plsc.VectorSubcoreMesh(core_axis_name="c", subcore_axis_name="s")


@functools.partial(
    pl.kernel,
    mesh=_sc_mesh,
    out_type=[
        jax.ShapeDtypeStruct((B, 8, 128), jnp.int32),
        jax.ShapeDtypeStruct((B, 8, 128), jnp.int32),
    ],
    scratch_types=[
        pltpu.VMEM((B_PER_W,), jnp.int32),
        pltpu.VMEM((B_PER_W,), jnp.int32),
        pltpu.VMEM((CH, 8, 128), jnp.int32),
        pltpu.VMEM((CH, 8, 128), jnp.int32),
        pltpu.SemaphoreType.DMA,
        pltpu.SemaphoreType.DMA,
    ],
)
def _sc_gather2(gw_hbm, gc_hbm, wtab_hbm, ctab_hbm, ew_hbm, ec_hbm,
                idx_w, idx_c, rows_w, rows_c, sem_w, sem_c):
    wid = lax.axis_index("s") * NC + lax.axis_index("c")
    base = wid * B_PER_W
    pltpu.sync_copy(gw_hbm.at[pl.ds(base, B_PER_W)], idx_w)
    pltpu.sync_copy(gc_hbm.at[pl.ds(base, B_PER_W)], idx_c)
    for h in range(B_PER_W // CH):
        cp_w = pltpu.async_copy(
            wtab_hbm.at[idx_w.at[pl.ds(h * CH, CH)]], rows_w, sem_w)
        cp_c = pltpu.async_copy(
            ctab_hbm.at[idx_c.at[pl.ds(h * CH, CH)]], rows_c, sem_c)
        cp_w.wait()
        cp_c.wait()
        pltpu.sync_copy(rows_w, ew_hbm.at[pl.ds(base + h * CH, CH)])
        pltpu.sync_copy(rows_c, ec_hbm.at[pl.ds(base + h * CH, CH)])


SB = 512


def _sel_body(ew32_ref, ec32_ref, selw_ref, selc_ref, ew_ref, ec_ref):
    lgrp = 2 * (lax.broadcasted_iota(jnp.int32, (SB, 128), 1) // 64)

    def pick(x_ref, sel_ref):
        sel = sel_ref[...]
        acc = jnp.zeros((SB, 128), jnp.float32)
        hi_mask = jnp.int32(-65536)
        for u in range(8):
            y = x_ref[:, u, :]
            f_lo = lax.bitcast_convert_type(
                lax.shift_left(y, 16), jnp.float32)
            f_hi = lax.bitcast_convert_type(
                lax.bitwise_and(y, hi_mask), jnp.float32)
            q = 4 * u + lgrp
            m_lo = (sel == q).astype(jnp.float32)
            m_hi = (sel == (q + 1)).astype(jnp.float32)
            acc += f_lo * m_lo + f_hi * m_hi
        return acc[:, 0:64] + acc[:, 64:128]

    ew_ref[...] = pick(ew32_ref, selw_ref)
    ec_ref[...] = pick(ec32_ref, selc_ref)


_select = pl.pallas_call(
    _sel_body,
    grid=(B // SB,),
    in_specs=[
        pl.BlockSpec((SB, 8, 128), lambda i: (i, 0, 0)),
        pl.BlockSpec((SB, 8, 128), lambda i: (i, 0, 0)),
        pl.BlockSpec((SB, 1), lambda i: (i, 0)),
        pl.BlockSpec((SB, 1), lambda i: (i, 0)),
    ],
    out_specs=[
        pl.BlockSpec((SB, D), lambda i: (i, 0)),
        pl.BlockSpec((SB, D), lambda i: (i, 0)),
    ],
    out_shape=[
        jax.ShapeDtypeStruct((B, D), jnp.float32),
        jax.ShapeDtypeStruct((B, D), jnp.float32),
    ],
)


CB = 512


def _score_body(ec_ref, ew_ref, o_ref):
    i = pl.program_id(0)
    s = lax.dot_general(
        ec_ref[...], ew_ref[...],
        dimension_numbers=(((1,), (1,)), ((), ())),
        preferred_element_type=jnp.float32,
    )
    val = jnp.sum(jnp.maximum(-s, 0.0) + jnp.log1p(jnp.exp(-jnp.abs(s))))

    @pl.when(i == 0)
    def _init():
        o_ref[0, 0] = 0.0

    o_ref[0, 0] += val


_score = pl.pallas_call(
    _score_body,
    grid=(B // CB,),
    in_specs=[
        pl.BlockSpec((CB, D), lambda i: (i, 0)),
        pl.BlockSpec((B, D), lambda i: (0, 0)),
    ],
    out_specs=pl.BlockSpec(
        (1, 1), lambda i: (0, 0), memory_space=pltpu.SMEM),
    out_shape=jax.ShapeDtypeStruct((1, 1), jnp.float32),
)


def kernel(node, context_positions, word_embedding, context_embedding):
    wtab = _pack(word_embedding.T)
    ctab = _pack(context_embedding.T)
    g_w = node // GR
    g_c = context_positions // GR
    sel_w = (node % GR).reshape(B, 1)
    sel_c = (context_positions % GR).reshape(B, 1)
    ew32, ec32 = _sc_gather2(g_w, g_c, wtab, ctab)
    embed_word, embed_context = _select(ew32, ec32, sel_w, sel_c)
    obj = _score(embed_context, embed_word)
    return obj[0, 0]


# speedup vs baseline: 1.8910x; 1.2440x over previous
import functools

import jax
import jax.numpy as jnp
from jax import lax
from jax.experimental import pallas as pl
from jax.experimental.pallas import tpu as pltpu
from jax.experimental.pallas import tpu_sc as plsc

NUM_NODES = 1000000
B = 4096
D = 64
GR = 32
LB = 4096
PG = NUM_NODES // LB + 1
NG_PAD = PG * (LB // GR)
NC = 2
NS = 16
NW = NC * NS
B_PER_W = B // NW
CH = 32


def _pack_body(wt_ref, ct_ref, wout_ref, cout_ref):
    def one(xt_ref, out_ref):
        yb = xt_ref[...].astype(jnp.bfloat16)
        y = lax.transpose(yb, (1, 0))
        wt = pltpu.bitcast(y, jnp.int32)
        w4 = wt.reshape(LB // GR, 8, 2, D)
        out_ref[:, :, 0:64] = w4[:, :, 0, :]
        out_ref[:, :, 64:128] = w4[:, :, 1, :]

    one(wt_ref, wout_ref)
    one(ct_ref, cout_ref)


_pack = pl.pallas_call(
    _pack_body,
    grid=(PG,),
    in_specs=[
        pl.BlockSpec((D, LB), lambda i: (0, i)),
        pl.BlockSpec((D, LB), lambda i: (0, i)),
    ],
    out_specs=[
        pl.BlockSpec((LB // GR, 8, 128), lambda i: (i, 0, 0)),
        pl.BlockSpec((LB // GR, 8, 128), lambda i: (i, 0, 0)),
    ],
    out_shape=[
        jax.ShapeDtypeStruct((NG_PAD, 8, 128), jnp.int32),
        jax.ShapeDtypeStruct((NG_PAD, 8, 128), jnp.int32),
    ],
)


_sc_mesh = plsc.VectorSubcoreMesh(core_axis_name="c", subcore_axis_name="s")


@functools.partial(
    pl.kernel,
    mesh=_sc_mesh,
    out_type=[
        jax.ShapeDtypeStruct((B, 8, 128), jnp.int32),
        jax.ShapeDtypeStruct((B, 8, 128), jnp.int32),
    ],
    scratch_types=[
        pltpu.VMEM((B_PER_W,), jnp.int32),
        pltpu.VMEM((B_PER_W,), jnp.int32),
        pltpu.VMEM((CH, 8, 128), jnp.int32),
        pltpu.VMEM((CH, 8, 128), jnp.int32),
        pltpu.SemaphoreType.DMA,
        pltpu.SemaphoreType.DMA,
    ],
)
def _sc_gather2(gw_hbm, gc_hbm, wtab_hbm, ctab_hbm, ew_hbm, ec_hbm,
                idx_w, idx_c, rows_w, rows_c, sem_w, sem_c):
    wid = lax.axis_index("s") * NC + lax.axis_index("c")
    base = wid * B_PER_W
    pltpu.sync_copy(gw_hbm.at[pl.ds(base, B_PER_W)], idx_w)
    pltpu.sync_copy(gc_hbm.at[pl.ds(base, B_PER_W)], idx_c)
    for h in range(B_PER_W // CH):
        cp_w = pltpu.async_copy(
            wtab_hbm.at[idx_w.at[pl.ds(h * CH, CH)]], rows_w, sem_w)
        cp_c = pltpu.async_copy(
            ctab_hbm.at[idx_c.at[pl.ds(h * CH, CH)]], rows_c, sem_c)
        cp_w.wait()
        cp_c.wait()
        pltpu.sync_copy(rows_w, ew_hbm.at[pl.ds(base + h * CH, CH)])
        pltpu.sync_copy(rows_c, ec_hbm.at[pl.ds(base + h * CH, CH)])


SB = 512


def _sel_body(ew32_ref, ec32_ref, selw_ref, selc_ref, ew_ref, ec_ref):
    lgrp = 2 * (lax.broadcasted_iota(jnp.int32, (SB, 128), 1) // 64)

    def pick(x_ref, sel_ref):
        sel = sel_ref[...]
        acc = jnp.zeros((SB, 128), jnp.float32)
        hi_mask = jnp.int32(-65536)
        for u in range(8):
            y = x_ref[:, u, :]
            f_lo = lax.bitcast_convert_type(
                lax.shift_left(y, 16), jnp.float32)
            f_hi = lax.bitcast_convert_type(
                lax.bitwise_and(y, hi_mask), jnp.float32)
            q = 4 * u + lgrp
            m_lo = (sel == q).astype(jnp.float32)
            m_hi = (sel == (q + 1)).astype(jnp.float32)
            acc += f_lo * m_lo + f_hi * m_hi
        return acc[:, 0:64] + acc[:, 64:128]

    ew_ref[...] = pick(ew32_ref, selw_ref)
    ec_ref[...] = pick(ec32_ref, selc_ref)


_select = pl.pallas_call(
    _sel_body,
    grid=(B // SB,),
    in_specs=[
        pl.BlockSpec((SB, 8, 128), lambda i: (i, 0, 0)),
        pl.BlockSpec((SB, 8, 128), lambda i: (i, 0, 0)),
        pl.BlockSpec((SB, 1), lambda i: (i, 0)),
        pl.BlockSpec((SB, 1), lambda i: (i, 0)),
    ],
    out_specs=[
        pl.BlockSpec((SB, D), lambda i: (i, 0)),
        pl.BlockSpec((SB, D), lambda i: (i, 0)),
    ],
    out_shape=[
        jax.ShapeDtypeStruct((B, D), jnp.float32),
        jax.ShapeDtypeStruct((B, D), jnp.float32),
    ],
)


CB = 512


def _score_body(ec_ref, ew_ref, o_ref):
    i = pl.program_id(0)
    s = lax.dot_general(
        ec_ref[...], ew_ref[...],
        dimension_numbers=(((1,), (1,)), ((), ())),
        preferred_element_type=jnp.float32,
    )
    val = jnp.sum(jnp.maximum(-s, 0.0) + jnp.log1p(jnp.exp(-jnp.abs(s))))

    @pl.when(i == 0)
    def _init():
        o_ref[0, 0] = 0.0

    o_ref[0, 0] += val


_score = pl.pallas_call(
    _score_body,
    grid=(B // CB,),
    in_specs=[
        pl.BlockSpec((CB, D), lambda i: (i, 0)),
        pl.BlockSpec((B, D), lambda i: (0, 0)),
    ],
    out_specs=pl.BlockSpec(
        (1, 1), lambda i: (0, 0), memory_space=pltpu.SMEM),
    out_shape=jax.ShapeDtypeStruct((1, 1), jnp.float32),
)


def kernel(node, context_positions, word_embedding, context_embedding):
    wtab, ctab = _pack(word_embedding.T, context_embedding.T)
    g_w = node // GR
    g_c = context_positions // GR
    sel_w = (node % GR).reshape(B, 1)
    sel_c = (context_positions % GR).reshape(B, 1)
    ew32, ec32 = _sc_gather2(g_w, g_c, wtab, ctab)
    embed_word, embed_context = _select(ew32, ec32, sel_w, sel_c)
    obj = _score(embed_context, embed_word)
    return obj[0, 0]


# speedup vs baseline: 2.0053x; 1.0604x over previous
import functools

import jax
import jax.numpy as jnp
from jax import lax
from jax.experimental import pallas as pl
from jax.experimental.pallas import tpu as pltpu
from jax.experimental.pallas import tpu_sc as plsc

NUM_NODES = 1000000
B = 4096
D = 64
GR = 32
LB = 8192
PG = NUM_NODES // LB + 1
NG_PAD = PG * (LB // GR)
NC = 2
NS = 16
NW = NC * NS
B_PER_W = B // NW
CH = 32


def _pack_body(wt_ref, ct_ref, wout_ref, cout_ref):
    def one(xt_ref, out_ref):
        yb = xt_ref[...].astype(jnp.bfloat16)
        y = lax.transpose(yb, (1, 0))
        wt = pltpu.bitcast(y, jnp.int32)
        w4 = wt.reshape(LB // GR, 8, 2, D)
        out_ref[:, :, 0:64] = w4[:, :, 0, :]
        out_ref[:, :, 64:128] = w4[:, :, 1, :]

    one(wt_ref, wout_ref)
    one(ct_ref, cout_ref)


_pack = pl.pallas_call(
    _pack_body,
    grid=(PG,),
    in_specs=[
        pl.BlockSpec((D, LB), lambda i: (0, i)),
        pl.BlockSpec((D, LB), lambda i: (0, i)),
    ],
    out_specs=[
        pl.BlockSpec((LB // GR, 8, 128), lambda i: (i, 0, 0)),
        pl.BlockSpec((LB // GR, 8, 128), lambda i: (i, 0, 0)),
    ],
    out_shape=[
        jax.ShapeDtypeStruct((NG_PAD, 8, 128), jnp.int32),
        jax.ShapeDtypeStruct((NG_PAD, 8, 128), jnp.int32),
    ],
)


_sc_mesh = plsc.VectorSubcoreMesh(core_axis_name="c", subcore_axis_name="s")


@functools.partial(
    pl.kernel,
    mesh=_sc_mesh,
    out_type=[
        jax.ShapeDtypeStruct((B, 8, 128), jnp.int32),
        jax.ShapeDtypeStruct((B, 8, 128), jnp.int32),
    ],
    scratch_types=[
        pltpu.VMEM((B_PER_W,), jnp.int32),
        pltpu.VMEM((B_PER_W,), jnp.int32),
        pltpu.VMEM((CH, 8, 128), jnp.int32),
        pltpu.VMEM((CH, 8, 128), jnp.int32),
        pltpu.SemaphoreType.DMA,
        pltpu.SemaphoreType.DMA,
    ],
)
def _sc_gather2(gw_hbm, gc_hbm, wtab_hbm, ctab_hbm, ew_hbm, ec_hbm,
                idx_w, idx_c, rows_w, rows_c, sem_w, sem_c):
    wid = lax.axis_index("s") * NC + lax.axis_index("c")
    base = wid * B_PER_W
    pltpu.sync_copy(gw_hbm.at[pl.ds(base, B_PER_W)], idx_w)
    pltpu.sync_copy(gc_hbm.at[pl.ds(base, B_PER_W)], idx_c)
    for h in range(B_PER_W // CH):
        cp_w = pltpu.async_copy(
            wtab_hbm.at[idx_w.at[pl.ds(h * CH, CH)]], rows_w, sem_w)
        cp_c = pltpu.async_copy(
            ctab_hbm.at[idx_c.at[pl.ds(h * CH, CH)]], rows_c, sem_c)
        cp_w.wait()
        cp_c.wait()
        pltpu.sync_copy(rows_w, ew_hbm.at[pl.ds(base + h * CH, CH)])
        pltpu.sync_copy(rows_c, ec_hbm.at[pl.ds(base + h * CH, CH)])


SB = 512


def _sel_body(ew32_ref, ec32_ref, selw_ref, selc_ref, ew_ref, ec_ref):
    lgrp = 2 * (lax.broadcasted_iota(jnp.int32, (SB, 128), 1) // 64)

    def pick(x_ref, sel_ref):
        sel = sel_ref[...]
        acc = jnp.zeros((SB, 128), jnp.float32)
        hi_mask = jnp.int32(-65536)
        for u in range(8):
            y = x_ref[:, u, :]
            f_lo = lax.bitcast_convert_type(
                lax.shift_left(y, 16), jnp.float32)
            f_hi = lax.bitcast_convert_type(
                lax.bitwise_and(y, hi_mask), jnp.float32)
            q = 4 * u + lgrp
            m_lo = (sel == q).astype(jnp.float32)
            m_hi = (sel == (q + 1)).astype(jnp.float32)
            acc += f_lo * m_lo + f_hi * m_hi
        return acc[:, 0:64] + acc[:, 64:128]

    ew_ref[...] = pick(ew32_ref, selw_ref)
    ec_ref[...] = pick(ec32_ref, selc_ref)


_select = pl.pallas_call(
    _sel_body,
    grid=(B // SB,),
    in_specs=[
        pl.BlockSpec((SB, 8, 128), lambda i: (i, 0, 0)),
        pl.BlockSpec((SB, 8, 128), lambda i: (i, 0, 0)),
        pl.BlockSpec((SB, 1), lambda i: (i, 0)),
        pl.BlockSpec((SB, 1), lambda i: (i, 0)),
    ],
    out_specs=[
        pl.BlockSpec((SB, D), lambda i: (i, 0)),
        pl.BlockSpec((SB, D), lambda i: (i, 0)),
    ],
    out_shape=[
        jax.ShapeDtypeStruct((B, D), jnp.float32),
        jax.ShapeDtypeStruct((B, D), jnp.float32),
    ],
)


CB = 512


def _score_body(ec_ref, ew_ref, o_ref):
    i = pl.program_id(0)
    s = lax.dot_general(
        ec_ref[...], ew_ref[...],
        dimension_numbers=(((1,), (1,)), ((), ())),
        preferred_element_type=jnp.float32,
    )
    val = jnp.sum(jnp.maximum(-s, 0.0) + jnp.log1p(jnp.exp(-jnp.abs(s))))

    @pl.when(i == 0)
    def _init():
        o_ref[0, 0] = 0.0

    o_ref[0, 0] += val


_score = pl.pallas_call(
    _score_body,
    grid=(B // CB,),
    in_specs=[
        pl.BlockSpec((CB, D), lambda i: (i, 0)),
        pl.BlockSpec((B, D), lambda i: (0, 0)),
    ],
    out_specs=pl.BlockSpec(
        (1, 1), lambda i: (0, 0), memory_space=pltpu.SMEM),
    out_shape=jax.ShapeDtypeStruct((1, 1), jnp.float32),
)


def kernel(node, context_positions, word_embedding, context_embedding):
    wtab, ctab = _pack(word_embedding.T, context_embedding.T)
    g_w = node // GR
    g_c = context_positions // GR
    sel_w = (node % GR).reshape(B, 1)
    sel_c = (context_positions % GR).reshape(B, 1)
    ew32, ec32 = _sc_gather2(g_w, g_c, wtab, ctab)
    embed_word, embed_context = _select(ew32, ec32, sel_w, sel_c)
    obj = _score(embed_context, embed_word)
    return obj[0, 0]
